# baseline (speedup 1.0000x reference)
import jax
import jax.numpy as jnp
from jax.experimental import pallas as pl
from jax.experimental.pallas import tpu as pltpu

L = 4
S = 4
N_MAX = 10
HID = 32
R_CUT = 5.0
FEAT = S * N_MAX
SH = S * HID

BLOCK = 2000


def _rb_mlp_kernel(r_ref, sp_ref, w1_ref, w2_ref, w3_ref, w4_ref, out_ref):
    r = r_ref[...]
    sp = sp_ref[...]
    b = r.shape[0]
    r_ = r * (1.0 / R_CUT)

    lane = jax.lax.broadcasted_iota(jnp.int32, (b, FEAT), 1)
    maskf = ((lane // N_MAX) == sp).astype(jnp.float32)
    nidx = (lane % N_MAX).astype(jnp.float32)

    r2 = r_ * r_
    pow_l = (None, r_, r2, r2 * r_)

    def silu(x):
        return x * jax.lax.logistic(x)

    for l in range(L):
        z = jnp.pi * (nidx + (1.0 + 0.5 * l))
        x = z * r_
        sinc = jnp.sin(x) / jnp.maximum(x, 1e-6)
        env = sinc if l == 0 else sinc * pow_l[l]
        xb = env * maskf
        h = silu(jnp.dot(xb, w1_ref[l], preferred_element_type=jnp.float32))
        h = silu(jnp.dot(h, w2_ref[l], preferred_element_type=jnp.float32))
        h = silu(jnp.dot(h, w3_ref[l], preferred_element_type=jnp.float32))
        y = jnp.dot(h, w4_ref[l], preferred_element_type=jnp.float32)
        out_ref[l] = y


@jax.jit
def kernel(r, species_neighbor, W1, W2, W3, W4):
    n = r.shape[0]
    block = BLOCK
    grid = n // block

    w1b = jnp.zeros((L, FEAT, SH), jnp.float32)
    w2b = jnp.zeros((L, SH, SH), jnp.float32)
    w3b = jnp.zeros((L, SH, SH), jnp.float32)
    for s in range(S):
        w1b = w1b.at[:, s * N_MAX:(s + 1) * N_MAX, s * HID:(s + 1) * HID].set(W1[:, s])
        w2b = w2b.at[:, s * HID:(s + 1) * HID, s * HID:(s + 1) * HID].set(W2[:, s])
        w3b = w3b.at[:, s * HID:(s + 1) * HID, s * HID:(s + 1) * HID].set(W3[:, s])
    w4r = W4.reshape(L, S * HID, N_MAX)

    r2d = r.reshape(n, 1)
    sp2d = species_neighbor.reshape(n, 1)

    return pl.pallas_call(
        _rb_mlp_kernel,
        grid=(grid,),
        in_specs=[
            pl.BlockSpec((block, 1), lambda i: (i, 0)),
            pl.BlockSpec((block, 1), lambda i: (i, 0)),
            pl.BlockSpec((L, FEAT, SH), lambda i: (0, 0, 0)),
            pl.BlockSpec((L, SH, SH), lambda i: (0, 0, 0)),
            pl.BlockSpec((L, SH, SH), lambda i: (0, 0, 0)),
            pl.BlockSpec((L, SH, N_MAX), lambda i: (0, 0, 0)),
        ],
        out_specs=pl.BlockSpec((L, block, N_MAX), lambda i: (0, i, 0)),
        out_shape=jax.ShapeDtypeStruct((L, n, N_MAX), jnp.float32),
        compiler_params=pltpu.CompilerParams(
            dimension_semantics=("arbitrary",),
        ),
    )(r2d, sp2d, w1b, w2b, w3b, w4r)


# speedup vs baseline: 2.2686x; 2.2686x over previous
import jax
import jax.numpy as jnp
from jax.experimental import pallas as pl
from jax.experimental.pallas import tpu as pltpu

L = 4
S = 4
N_MAX = 10
HID = 32
R_CUT = 5.0
FEAT = S * N_MAX
SH = S * HID

BLOCK = 2000


def _fast_sin(x):
    n = jnp.floor(x * (2.0 / jnp.pi) + 0.5)
    y = x - n * (jnp.pi / 2.0)
    q = n - 4.0 * jnp.floor(n * 0.25)
    y2 = y * y
    sin_p = y * (1.0 + y2 * (-1.6666667e-1 + y2 * (8.3333310e-3 + y2 * -1.98409e-4)))
    cos_p = 1.0 + y2 * (-0.5 + y2 * (4.16666418e-2 + y2 * -1.388731625e-3))
    use_cos = jnp.logical_or(q == 1.0, q == 3.0)
    val = jnp.where(use_cos, cos_p, sin_p)
    return jnp.where(q >= 2.0, -val, val)


def _rb_mlp_kernel(r_ref, sp_ref, w1_ref, w2_ref, w3_ref, w4_ref, out_ref):
    r = r_ref[...]
    sp = sp_ref[...]
    b = r.shape[0]
    r_ = r * (1.0 / R_CUT)

    lane = jax.lax.broadcasted_iota(jnp.int32, (b, FEAT), 1)
    maskf = ((lane // N_MAX) == sp).astype(jnp.float32)
    l_id = lane // N_MAX
    nidx = (lane % N_MAX).astype(jnp.float32)

    z = jnp.pi * (nidx + 1.0) + (jnp.pi * 0.5) * l_id.astype(jnp.float32)
    x = z * r_
    sinc = _fast_sin(x) / jnp.maximum(x, 1e-6)
    r2 = r_ * r_
    env = jnp.where(l_id == 0, 1.0,
          jnp.where(l_id == 1, r_,
          jnp.where(l_id == 2, r2, r2 * r_)))
    rf = sinc * env

    def silu(v):
        return v * jax.lax.logistic(v)

    for l in range(L):
        rf_l = jax.lax.slice_in_dim(rf, l * N_MAX, (l + 1) * N_MAX, axis=1)
        xb = jnp.concatenate([rf_l, rf_l, rf_l, rf_l], axis=1) * maskf
        h = silu(jnp.dot(xb, w1_ref[l], preferred_element_type=jnp.float32))
        h = silu(jnp.dot(h, w2_ref[l], preferred_element_type=jnp.float32))
        h = silu(jnp.dot(h, w3_ref[l], preferred_element_type=jnp.float32))
        y = jnp.dot(h, w4_ref[l], preferred_element_type=jnp.float32)
        out_ref[l] = y


@jax.jit
def kernel(r, species_neighbor, W1, W2, W3, W4):
    n = r.shape[0]
    block = BLOCK
    grid = n // block

    w1b = jnp.zeros((L, FEAT, SH), jnp.float32)
    w2b = jnp.zeros((L, SH, SH), jnp.float32)
    w3b = jnp.zeros((L, SH, SH), jnp.float32)
    for s in range(S):
        w1b = w1b.at[:, s * N_MAX:(s + 1) * N_MAX, s * HID:(s + 1) * HID].set(W1[:, s])
        w2b = w2b.at[:, s * HID:(s + 1) * HID, s * HID:(s + 1) * HID].set(W2[:, s])
        w3b = w3b.at[:, s * HID:(s + 1) * HID, s * HID:(s + 1) * HID].set(W3[:, s])
    w4r = W4.reshape(L, S * HID, N_MAX)

    r2d = r.reshape(n, 1)
    sp2d = species_neighbor.reshape(n, 1)

    return pl.pallas_call(
        _rb_mlp_kernel,
        grid=(grid,),
        in_specs=[
            pl.BlockSpec((block, 1), lambda i: (i, 0)),
            pl.BlockSpec((block, 1), lambda i: (i, 0)),
            pl.BlockSpec((L, FEAT, SH), lambda i: (0, 0, 0)),
            pl.BlockSpec((L, SH, SH), lambda i: (0, 0, 0)),
            pl.BlockSpec((L, SH, SH), lambda i: (0, 0, 0)),
            pl.BlockSpec((L, SH, N_MAX), lambda i: (0, 0, 0)),
        ],
        out_specs=pl.BlockSpec((L, block, N_MAX), lambda i: (0, i, 0)),
        out_shape=jax.ShapeDtypeStruct((L, n, N_MAX), jnp.float32),
        compiler_params=pltpu.CompilerParams(
            dimension_semantics=("arbitrary",),
        ),
    )(r2d, sp2d, w1b, w2b, w3b, w4r)


# speedup vs baseline: 2.5130x; 1.1077x over previous
import jax
import jax.numpy as jnp
from jax.experimental import pallas as pl
from jax.experimental.pallas import tpu as pltpu

L = 4
S = 4
N_MAX = 10
HID = 32
R_CUT = 5.0
FEAT = S * N_MAX
SH = S * HID

BLOCK = 2000


def _fast_sin(x):
    n = jnp.floor(x * (2.0 / jnp.pi) + 0.5)
    y = x - n * (jnp.pi / 2.0)
    q = n - 4.0 * jnp.floor(n * 0.25)
    y2 = y * y
    sin_p = y * (1.0 + y2 * (-1.6666667e-1 + y2 * (8.3333310e-3 + y2 * -1.98409e-4)))
    cos_p = 1.0 + y2 * (-0.5 + y2 * (4.16666418e-2 + y2 * -1.388731625e-3))
    use_cos = jnp.logical_or(q == 1.0, q == 3.0)
    val = jnp.where(use_cos, cos_p, sin_p)
    return jnp.where(q >= 2.0, -val, val)


def _rb_mlp_kernel(r_ref, sp_ref, w1_ref, w2_ref, w3_ref, w4_ref, out_ref):
    r = r_ref[...]
    sp = sp_ref[...]
    b = r.shape[0]
    r_ = r * (1.0 / R_CUT)

    lane = jax.lax.broadcasted_iota(jnp.int32, (b, FEAT), 1)
    maskf = ((lane // N_MAX) == sp).astype(jnp.float32)
    l_id = lane // N_MAX
    nidx = (lane % N_MAX).astype(jnp.float32)

    z = jnp.pi * (nidx + 1.0) + (jnp.pi * 0.5) * l_id.astype(jnp.float32)
    x = z * r_
    sinc = _fast_sin(x) / jnp.maximum(x, 1e-6)
    r2 = r_ * r_
    env = jnp.where(l_id == 0, 1.0,
          jnp.where(l_id == 1, r_,
          jnp.where(l_id == 2, r2, r2 * r_)))
    rf = sinc * env

    def silu_h(u):
        return u + u * jnp.tanh(u)

    for l in range(L):
        rf_l = jax.lax.slice_in_dim(rf, l * N_MAX, (l + 1) * N_MAX, axis=1)
        xb = jnp.concatenate([rf_l, rf_l, rf_l, rf_l], axis=1) * maskf
        h = silu_h(jnp.dot(xb, w1_ref[l], preferred_element_type=jnp.float32))
        h = silu_h(jnp.dot(h, w2_ref[l], preferred_element_type=jnp.float32))
        h = silu_h(jnp.dot(h, w3_ref[l], preferred_element_type=jnp.float32))
        y = jnp.dot(h, w4_ref[l], preferred_element_type=jnp.float32)
        out_ref[l] = y


@jax.jit
def kernel(r, species_neighbor, W1, W2, W3, W4):
    n = r.shape[0]
    block = BLOCK
    grid = n // block

    w1b = jnp.zeros((L, FEAT, SH), jnp.float32)
    w2b = jnp.zeros((L, SH, SH), jnp.float32)
    w3b = jnp.zeros((L, SH, SH), jnp.float32)
    for s in range(S):
        w1b = w1b.at[:, s * N_MAX:(s + 1) * N_MAX, s * HID:(s + 1) * HID].set(W1[:, s])
        w2b = w2b.at[:, s * HID:(s + 1) * HID, s * HID:(s + 1) * HID].set(W2[:, s])
        w3b = w3b.at[:, s * HID:(s + 1) * HID, s * HID:(s + 1) * HID].set(W3[:, s])
    w1b = w1b * 0.5
    w2b = w2b * 0.5
    w3b = w3b * 0.5
    w4r = W4.reshape(L, S * HID, N_MAX)

    r2d = r.reshape(n, 1)
    sp2d = species_neighbor.reshape(n, 1)

    return pl.pallas_call(
        _rb_mlp_kernel,
        grid=(grid,),
        in_specs=[
            pl.BlockSpec((block, 1), lambda i: (i, 0)),
            pl.BlockSpec((block, 1), lambda i: (i, 0)),
            pl.BlockSpec((L, FEAT, SH), lambda i: (0, 0, 0)),
            pl.BlockSpec((L, SH, SH), lambda i: (0, 0, 0)),
            pl.BlockSpec((L, SH, SH), lambda i: (0, 0, 0)),
            pl.BlockSpec((L, SH, N_MAX), lambda i: (0, 0, 0)),
        ],
        out_specs=pl.BlockSpec((L, block, N_MAX), lambda i: (0, i, 0)),
        out_shape=jax.ShapeDtypeStruct((L, n, N_MAX), jnp.float32),
        compiler_params=pltpu.CompilerParams(
            dimension_semantics=("arbitrary",),
        ),
    )(r2d, sp2d, w1b, w2b, w3b, w4r)
